# Initial kernel scaffold; baseline (speedup 1.0000x reference)
import jax
import jax.numpy as jnp
from jax.experimental import pallas as pl


def kernel(x, edge_indices, Wl0, bl0, Wr0, Wl1, bl1, Wr1, Wl2, bl2, Wr2):
    raise NotImplementedError("write your pallas kernel here")


# speedup vs baseline: 3.1988x; 3.1988x over previous
import functools

import jax
import jax.numpy as jnp
from jax import lax
from jax.experimental import pallas as pl
from jax.experimental.pallas import tpu as pltpu
from jax.experimental.pallas import tpu_sc as plsc

N = 10000
D = 128
R = 3
E = 320000
K = 128
NC = 2
NS = 16
NW = NC * NS
CH = 79
EPAD = NW * CH * K
NP = 10240
SEG = NP // NS
FCH = SEG // K
DEGW = 128


def _sc_segment_sums(x, srcs, dsts, zacc):
  mesh = plsc.VectorSubcoreMesh(core_axis_name="c", subcore_axis_name="s")

  @functools.partial(
      pl.kernel,
      out_type=jax.ShapeDtypeStruct((R, NC, NP, D), jnp.float32),
      mesh=mesh,
      scratch_types=[
          pltpu.VMEM_SHARED((NP, D), jnp.float32),
          pltpu.VMEM((K,), jnp.int32),
          pltpu.VMEM((K,), jnp.int32),
          pltpu.VMEM((K, D), jnp.float32),
          pltpu.SemaphoreType.DMA,
      ],
  )
  def k(x_hbm, src_hbm, dst_hbm, zacc_hbm,
        agg_hbm,
        acc_sh, src_v, dst_v, rows_v, sem):
    c = lax.axis_index("c")
    s = lax.axis_index("s")
    w = c * NS + s
    base = s * SEG

    for r in range(R):
      pltpu.sync_copy(zacc_hbm, rows_v)
      for t in range(FCH):
        pltpu.sync_copy(rows_v, acc_sh.at[pl.ds(base + t * K, K)])
      plsc.subcore_barrier()

      @pl.loop(0, CH)
      def _(t):
        e0 = r * EPAD + (w * CH + t) * K
        pltpu.sync_copy(src_hbm.at[pl.ds(e0, K)], src_v)
        pltpu.sync_copy(dst_hbm.at[pl.ds(e0, K)], dst_v)
        pltpu.async_copy(x_hbm.at[src_v], rows_v, sem).wait()
        pltpu.sync_copy(rows_v, acc_sh.at[dst_v], add=True)

      plsc.subcore_barrier()
      for t in range(FCH):
        pltpu.sync_copy(acc_sh.at[pl.ds(base + t * K, K)], rows_v)
        pltpu.sync_copy(rows_v, agg_hbm.at[r, c, pl.ds(base + t * K, K)])
      plsc.subcore_barrier()

  return k(x, srcs, dsts, zacc)


def _sc_degrees(dsts, zdeg, ones):
  mesh = plsc.VectorSubcoreMesh(core_axis_name="c", subcore_axis_name="s")

  @functools.partial(
      pl.kernel,
      out_type=jax.ShapeDtypeStruct((R, NC, NP, DEGW), jnp.float32),
      mesh=mesh,
      scratch_types=[
          pltpu.VMEM_SHARED((NP, DEGW), jnp.float32),
          pltpu.VMEM((K,), jnp.int32),
          pltpu.VMEM((K, DEGW), jnp.float32),
      ],
  )
  def k(dst_hbm, zdeg_hbm, ones_hbm,
        deg_hbm,
        deg_sh, dst_v, deg1_v):
    c = lax.axis_index("c")
    s = lax.axis_index("s")
    w = c * NS + s
    base = s * SEG

    for r in range(R):
      pltpu.sync_copy(zdeg_hbm, deg1_v)
      for t in range(FCH):
        pltpu.sync_copy(deg1_v, deg_sh.at[pl.ds(base + t * K, K)])
      pltpu.sync_copy(ones_hbm, deg1_v)
      plsc.subcore_barrier()

      @pl.loop(0, CH)
      def _(t):
        e0 = r * EPAD + (w * CH + t) * K
        pltpu.sync_copy(dst_hbm.at[pl.ds(e0, K)], dst_v)
        pltpu.sync_copy(deg1_v, deg_sh.at[dst_v], add=True)

      plsc.subcore_barrier()
      for t in range(FCH):
        pltpu.sync_copy(deg_sh.at[pl.ds(base + t * K, K)], deg1_v)
        pltpu.sync_copy(deg1_v, deg_hbm.at[r, c, pl.ds(base + t * K, K)])
      plsc.subcore_barrier()

  return k(dsts, zdeg, ones)


def _tc_combine(x, agg, deg, wlt, wrt, bls):
  B = 1000

  def body(x_ref, agg_ref, deg_ref, wlt_ref, wrt_ref, bl_ref, o_ref):
    acc = jnp.dot(x_ref[...], wrt_ref[...],
                  preferred_element_type=jnp.float32,
                  precision=lax.Precision.HIGHEST) + bl_ref[...]
    for r in range(R):
      aggr = agg_ref[r, 0] + agg_ref[r, 1]
      degr = deg_ref[r, 0, :, 0:1] + deg_ref[r, 1, :, 0:1]
      mean = aggr / jnp.maximum(degr, 1.0)
      acc = acc + jnp.dot(mean, wlt_ref[r],
                          preferred_element_type=jnp.float32,
                          precision=lax.Precision.HIGHEST)
    o_ref[...] = acc

  return pl.pallas_call(
      body,
      grid=(N // B,),
      in_specs=[
          pl.BlockSpec((B, D), lambda i: (i, 0)),
          pl.BlockSpec((R, NC, B, D), lambda i: (0, 0, i, 0)),
          pl.BlockSpec((R, NC, B, DEGW), lambda i: (0, 0, i, 0)),
          pl.BlockSpec((R, D, D), lambda i: (0, 0, 0)),
          pl.BlockSpec((D, D), lambda i: (0, 0)),
          pl.BlockSpec((1, D), lambda i: (0, 0)),
      ],
      out_specs=pl.BlockSpec((B, D), lambda i: (i, 0)),
      out_shape=jax.ShapeDtypeStruct((N, D), jnp.float32),
  )(x, agg, deg, wlt, wrt, bls)


def kernel(x, edge_indices, Wl0, bl0, Wr0, Wl1, bl1, Wr1, Wl2, bl2, Wr2):
  pad = EPAD - E
  srcs = jnp.pad(edge_indices[:, 0, :], ((0, 0), (0, pad))).reshape(R * EPAD)
  dsts = jnp.pad(edge_indices[:, 1, :], ((0, 0), (0, pad)),
                 constant_values=N).reshape(R * EPAD)
  zacc = jnp.zeros((K, D), jnp.float32)
  zdeg = jnp.zeros((K, DEGW), jnp.float32)
  ones = jnp.ones((K, DEGW), jnp.float32)
  agg = _sc_segment_sums(x, srcs, dsts, zacc)
  deg = _sc_degrees(dsts, zdeg, ones)
  wlt = jnp.stack([Wl0.T, Wl1.T, Wl2.T])
  wrt = (Wr0 + Wr1 + Wr2).T
  bls = (bl0 + bl1 + bl2).reshape(1, D)
  return _tc_combine(x, agg, deg, wlt, wrt, bls)
